# baseline (speedup 1.0000x reference)
import jax
import jax.numpy as jnp
from jax.experimental import pallas as pl
from jax.experimental.pallas import tpu as pltpu

HIDDEN_DIM = 2048
NUM_CHOICES = 64
BLOCK_M = 512


def _gate_kernel(x_ref, w_ref, b_ref, o_ref):
    logits = jax.lax.dot_general(
        x_ref[...], w_ref[...],
        dimension_numbers=(((1,), (1,)), ((), ())),
        preferred_element_type=jnp.float32,
    )
    logits = logits + b_ref[...]
    row_max = jnp.max(logits, axis=1, keepdims=True)
    col = jax.lax.broadcasted_iota(jnp.int32, logits.shape, 1)
    cand = jnp.where(logits == row_max, col, NUM_CHOICES)
    idx = jnp.min(cand, axis=1, keepdims=True)
    o_ref[...] = (col == idx).astype(o_ref.dtype)


def kernel(x, W, b):
    n = x.shape[0]
    b2 = b.reshape(1, NUM_CHOICES)
    return pl.pallas_call(
        _gate_kernel,
        grid=(n // BLOCK_M,),
        in_specs=[
            pl.BlockSpec((BLOCK_M, HIDDEN_DIM), lambda i: (i, 0)),
            pl.BlockSpec((NUM_CHOICES, HIDDEN_DIM), lambda i: (0, 0)),
            pl.BlockSpec((1, NUM_CHOICES), lambda i: (0, 0)),
        ],
        out_specs=pl.BlockSpec((BLOCK_M, NUM_CHOICES), lambda i: (i, 0)),
        out_shape=jax.ShapeDtypeStruct((n, NUM_CHOICES), x.dtype),
        compiler_params=pltpu.CompilerParams(
            dimension_semantics=("parallel",),
        ),
    )(x, W, b2)


# speedup vs baseline: 1.1805x; 1.1805x over previous
import jax
import jax.numpy as jnp
from jax.experimental import pallas as pl
from jax.experimental.pallas import tpu as pltpu

HIDDEN_DIM = 2048
NUM_CHOICES = 64
BLOCK_M = 1024


def _gate_kernel(x_ref, w_ref, b_ref, o_ref):
    logits = jax.lax.dot_general(
        x_ref[...], w_ref[...],
        dimension_numbers=(((1,), (1,)), ((), ())),
        preferred_element_type=jnp.float32,
    )
    logits = logits + b_ref[...]
    row_max = jnp.max(logits, axis=1, keepdims=True)
    col = jax.lax.broadcasted_iota(jnp.int32, logits.shape, 1)
    cand = jnp.where(logits == row_max, col, NUM_CHOICES)
    idx = jnp.min(cand, axis=1, keepdims=True)
    o_ref[...] = (col == idx).astype(o_ref.dtype)


def kernel(x, W, b):
    n = x.shape[0]
    b2 = b.reshape(1, NUM_CHOICES)
    return pl.pallas_call(
        _gate_kernel,
        grid=(n // BLOCK_M,),
        in_specs=[
            pl.BlockSpec((BLOCK_M, HIDDEN_DIM), lambda i: (i, 0)),
            pl.BlockSpec((NUM_CHOICES, HIDDEN_DIM), lambda i: (0, 0)),
            pl.BlockSpec((1, NUM_CHOICES), lambda i: (0, 0)),
        ],
        out_specs=pl.BlockSpec((BLOCK_M, NUM_CHOICES), lambda i: (i, 0)),
        out_shape=jax.ShapeDtypeStruct((n, NUM_CHOICES), x.dtype),
        compiler_params=pltpu.CompilerParams(
            dimension_semantics=("parallel",),
        ),
    )(x, W, b2)


# speedup vs baseline: 1.1852x; 1.0039x over previous
import jax
import jax.numpy as jnp
from jax.experimental import pallas as pl
from jax.experimental.pallas import tpu as pltpu

HIDDEN_DIM = 2048
NUM_CHOICES = 64
BLOCK_M = 2048


def _gate_kernel(x_ref, w_ref, b_ref, o_ref):
    logits = jax.lax.dot_general(
        x_ref[...], w_ref[...],
        dimension_numbers=(((1,), (1,)), ((), ())),
        preferred_element_type=jnp.float32,
    )
    logits = logits + b_ref[...]
    row_max = jnp.max(logits, axis=1, keepdims=True)
    col = jax.lax.broadcasted_iota(jnp.int32, logits.shape, 1)
    cand = jnp.where(logits == row_max, col, NUM_CHOICES)
    idx = jnp.min(cand, axis=1, keepdims=True)
    o_ref[...] = (col == idx).astype(o_ref.dtype)


def kernel(x, W, b):
    n = x.shape[0]
    b2 = b.reshape(1, NUM_CHOICES)
    return pl.pallas_call(
        _gate_kernel,
        grid=(n // BLOCK_M,),
        in_specs=[
            pl.BlockSpec((BLOCK_M, HIDDEN_DIM), lambda i: (i, 0)),
            pl.BlockSpec((NUM_CHOICES, HIDDEN_DIM), lambda i: (0, 0)),
            pl.BlockSpec((1, NUM_CHOICES), lambda i: (0, 0)),
        ],
        out_specs=pl.BlockSpec((BLOCK_M, NUM_CHOICES), lambda i: (i, 0)),
        out_shape=jax.ShapeDtypeStruct((n, NUM_CHOICES), x.dtype),
        compiler_params=pltpu.CompilerParams(
            dimension_semantics=("parallel",),
        ),
    )(x, W, b2)


# speedup vs baseline: 1.1938x; 1.0073x over previous
import jax
import jax.numpy as jnp
from jax.experimental import pallas as pl
from jax.experimental.pallas import tpu as pltpu

HIDDEN_DIM = 2048
NUM_CHOICES = 64
BLOCK_M = 2048
HALF_M = BLOCK_M // 2


def _half_onehot(x_half, w, b, o_ref, row0):
    logits = jax.lax.dot_general(
        x_half, w,
        dimension_numbers=(((1,), (1,)), ((), ())),
        preferred_element_type=jnp.float32,
    )
    logits = logits + b
    row_max = jnp.max(logits, axis=1, keepdims=True)
    col = jax.lax.broadcasted_iota(jnp.int32, logits.shape, 1)
    cand = jnp.where(logits == row_max, col, NUM_CHOICES)
    idx = jnp.min(cand, axis=1, keepdims=True)
    o_ref[pl.ds(row0, HALF_M), :] = (col == idx).astype(o_ref.dtype)


def _gate_kernel(x0_ref, x1_ref, w_ref, b_ref, o_ref):
    w = w_ref[...]
    b = b_ref[...]
    _half_onehot(x0_ref[...], w, b, o_ref, 0)
    _half_onehot(x1_ref[...], w, b, o_ref, HALF_M)


def kernel(x, W, b):
    n = x.shape[0]
    b2 = b.reshape(1, NUM_CHOICES)
    return pl.pallas_call(
        _gate_kernel,
        grid=(n // BLOCK_M,),
        in_specs=[
            pl.BlockSpec((HALF_M, HIDDEN_DIM), lambda i: (2 * i, 0)),
            pl.BlockSpec((HALF_M, HIDDEN_DIM), lambda i: (2 * i + 1, 0)),
            pl.BlockSpec((NUM_CHOICES, HIDDEN_DIM), lambda i: (0, 0)),
            pl.BlockSpec((1, NUM_CHOICES), lambda i: (0, 0)),
        ],
        out_specs=pl.BlockSpec((BLOCK_M, NUM_CHOICES), lambda i: (i, 0)),
        out_shape=jax.ShapeDtypeStruct((n, NUM_CHOICES), x.dtype),
        compiler_params=pltpu.CompilerParams(
            dimension_semantics=("parallel",),
        ),
    )(x, x, W, b2)


# speedup vs baseline: 1.3675x; 1.1454x over previous
import jax
import jax.numpy as jnp
from jax.experimental import pallas as pl
from jax.experimental.pallas import tpu as pltpu

HIDDEN_DIM = 2048
NUM_CHOICES = 64
BLOCK_M = 2048


def _gate_kernel(x_ref, w_ref, b_ref, o_ref):
    logits_t = jax.lax.dot_general(
        w_ref[...], x_ref[...],
        dimension_numbers=(((1,), (1,)), ((), ())),
        preferred_element_type=jnp.float32,
    )
    logits_t = logits_t + b_ref[...]
    col_max = jnp.max(logits_t, axis=0, keepdims=True)
    row = jax.lax.broadcasted_iota(jnp.int32, logits_t.shape, 0)
    cand = jnp.where(logits_t == col_max, row, NUM_CHOICES)
    idx = jnp.min(cand, axis=0, keepdims=True)
    o_ref[...] = (row == idx).astype(o_ref.dtype)


def kernel(x, W, b):
    n = x.shape[0]
    b2 = b.reshape(NUM_CHOICES, 1)
    out_t = pl.pallas_call(
        _gate_kernel,
        grid=(n // BLOCK_M,),
        in_specs=[
            pl.BlockSpec((BLOCK_M, HIDDEN_DIM), lambda i: (i, 0)),
            pl.BlockSpec((NUM_CHOICES, HIDDEN_DIM), lambda i: (0, 0)),
            pl.BlockSpec((NUM_CHOICES, 1), lambda i: (0, 0)),
        ],
        out_specs=pl.BlockSpec((NUM_CHOICES, BLOCK_M), lambda i: (0, i)),
        out_shape=jax.ShapeDtypeStruct((NUM_CHOICES, n), x.dtype),
        compiler_params=pltpu.CompilerParams(
            dimension_semantics=("arbitrary",),
        ),
    )(x, W, b2)
    return out_t.T


# speedup vs baseline: 1.3888x; 1.0156x over previous
import jax
import jax.numpy as jnp
from jax.experimental import pallas as pl
from jax.experimental.pallas import tpu as pltpu

HIDDEN_DIM = 2048
NUM_CHOICES = 64
BLOCK_M = 1024


def _gate_kernel(x_ref, w_ref, b_ref, o_ref):
    logits_t = jax.lax.dot_general(
        w_ref[...], x_ref[...],
        dimension_numbers=(((1,), (1,)), ((), ())),
        preferred_element_type=jnp.float32,
    )
    logits_t = logits_t + b_ref[...]
    col_max = jnp.max(logits_t, axis=0, keepdims=True)
    row = jax.lax.broadcasted_iota(jnp.int32, logits_t.shape, 0)
    cand = jnp.where(logits_t == col_max, row, NUM_CHOICES)
    idx = jnp.min(cand, axis=0, keepdims=True)
    o_ref[...] = (row == idx).astype(o_ref.dtype)


def kernel(x, W, b):
    n = x.shape[0]
    b2 = b.reshape(NUM_CHOICES, 1)
    out_t = pl.pallas_call(
        _gate_kernel,
        grid=(n // BLOCK_M,),
        in_specs=[
            pl.BlockSpec((BLOCK_M, HIDDEN_DIM), lambda i: (i, 0)),
            pl.BlockSpec((NUM_CHOICES, HIDDEN_DIM), lambda i: (0, 0)),
            pl.BlockSpec((NUM_CHOICES, 1), lambda i: (0, 0)),
        ],
        out_specs=pl.BlockSpec((NUM_CHOICES, BLOCK_M), lambda i: (0, i)),
        out_shape=jax.ShapeDtypeStruct((NUM_CHOICES, n), x.dtype),
        compiler_params=pltpu.CompilerParams(
            dimension_semantics=("arbitrary",),
        ),
    )(x, W, b2)
    return out_t.T
